# baseline (speedup 1.0000x reference)
import jax
import jax.numpy as jnp
from jax import lax
from jax.experimental import pallas as pl
from jax.experimental.pallas import tpu as pltpu
from jax.experimental.pallas import tpu_sc as plsc

B = 16384
L = 20
D = 64
N_TOTAL = B * L
NW = 32
ROWS_PER_W = B // NW
N_PER_W = N_TOTAL // NW
CHUNK = 128
N_CHUNKS = N_PER_W // CHUNK
NBUF = 4
N_STEPS = N_CHUNKS // NBUF
GPC = CHUNK // 16


def _sc_body(ci_hbm, xi_hbm, in_emb_hbm, out_emb_hbm, out_hbm,
             ci2d, xi2d, fci, fxi,
             cb0, xb0, cb1, xb1, cb2, xb2, cb3, xb3,
             scores, s0, s1, s2, s3):
    cbufs = (cb0, cb1, cb2, cb3)
    xbufs = (xb0, xb1, xb2, xb3)
    sems = (s0, s1, s2, s3)
    wid = lax.axis_index("s") * 2 + lax.axis_index("c")
    row_base = wid * ROWS_PER_W

    pltpu.sync_copy(ci_hbm.at[pl.ds(row_base, ROWS_PER_W)], ci2d)
    pltpu.sync_copy(xi_hbm.at[pl.ds(row_base, ROWS_PER_W)], xi2d)

    def repack(src2d, dst):
        def row_body(r, carry):
            dst[pl.ds(r * L, 16)] = src2d[r, pl.ds(0, 16)]
            dst[pl.ds(r * L + 4, 16)] = src2d[r, pl.ds(4, 16)]
            return carry
        lax.fori_loop(0, ROWS_PER_W, row_body, 0)

    repack(ci2d, fci)
    repack(xi2d, fxi)

    iota16 = lax.iota(jnp.int32, 16)

    def fire(c, k):
        pltpu.async_copy(in_emb_hbm.at[fci.at[pl.ds(c * CHUNK, CHUNK)]],
                         cbufs[k], sems[k])
        pltpu.async_copy(out_emb_hbm.at[fxi.at[pl.ds(c * CHUNK, CHUNK)]],
                         xbufs[k], sems[k])

    def drain(c, k):
        pltpu.make_async_copy(in_emb_hbm.at[fci.at[pl.ds(c * CHUNK, CHUNK)]],
                              cbufs[k], sems[k]).wait()
        pltpu.make_async_copy(out_emb_hbm.at[fxi.at[pl.ds(c * CHUNK, CHUNK)]],
                              xbufs[k], sems[k]).wait()

    def compute(c, k):
        cbuf, xbuf = cbufs[k], xbufs[k]

        def group_body(g, carry):
            base = g * 16
            acc = jnp.zeros((16,), jnp.float32)
            for r in range(16):
                row = base + r
                s = (cbuf[row, pl.ds(0, 16)] * xbuf[row, pl.ds(0, 16)]
                     + cbuf[row, pl.ds(16, 16)] * xbuf[row, pl.ds(16, 16)]
                     + cbuf[row, pl.ds(32, 16)] * xbuf[row, pl.ds(32, 16)]
                     + cbuf[row, pl.ds(48, 16)] * xbuf[row, pl.ds(48, 16)])
                acc = jnp.where(iota16 == r, jnp.sum(s), acc)
            scores[pl.ds(c * CHUNK + base, 16)] = acc
            return carry

        lax.fori_loop(0, GPC, group_body, 0)

    for k in range(NBUF):
        fire(k, k)

    def step_body(i, carry):
        for k in range(NBUF):
            c = i * NBUF + k
            drain(c, k)
            compute(c, k)

            @pl.when(i < N_STEPS - 1)
            def _():
                fire(c + NBUF, k)
        return carry

    lax.fori_loop(0, N_STEPS, step_body, 0)
    pltpu.sync_copy(scores, out_hbm.at[pl.ds(wid * N_PER_W, N_PER_W)])


def kernel(center_words, context_words, in_embeddings, out_embeddings):
    ci = center_words.astype(jnp.int32)
    xi = context_words.astype(jnp.int32)
    mesh = plsc.VectorSubcoreMesh(core_axis_name="c", subcore_axis_name="s")
    flat = pl.kernel(
        _sc_body,
        mesh=mesh,
        compiler_params=pltpu.CompilerParams(
            needs_layout_passes=False, use_tc_tiling_on_sc=False),
        out_type=jax.ShapeDtypeStruct((N_TOTAL,), jnp.float32),
        scratch_types=[
            pltpu.VMEM((ROWS_PER_W, L), jnp.int32),
            pltpu.VMEM((ROWS_PER_W, L), jnp.int32),
            pltpu.VMEM((N_PER_W,), jnp.int32),
            pltpu.VMEM((N_PER_W,), jnp.int32),
            pltpu.VMEM((CHUNK, D), jnp.float32),
            pltpu.VMEM((CHUNK, D), jnp.float32),
            pltpu.VMEM((CHUNK, D), jnp.float32),
            pltpu.VMEM((CHUNK, D), jnp.float32),
            pltpu.VMEM((CHUNK, D), jnp.float32),
            pltpu.VMEM((CHUNK, D), jnp.float32),
            pltpu.VMEM((CHUNK, D), jnp.float32),
            pltpu.VMEM((CHUNK, D), jnp.float32),
            pltpu.VMEM((N_PER_W,), jnp.float32),
            pltpu.SemaphoreType.DMA,
            pltpu.SemaphoreType.DMA,
            pltpu.SemaphoreType.DMA,
            pltpu.SemaphoreType.DMA,
        ],
    )(ci, xi, in_embeddings, out_embeddings)
    return flat.reshape(B, L)


# speedup vs baseline: 1.0025x; 1.0025x over previous
import jax
import jax.numpy as jnp
from jax import lax
from jax.experimental import pallas as pl
from jax.experimental.pallas import tpu as pltpu
from jax.experimental.pallas import tpu_sc as plsc

B = 16384
L = 20
LP = 128
D = 64
N_TOTAL = B * L
NW = 32
ROWS_PER_W = B // NW
N_PER_W = N_TOTAL // NW
CHUNK = 128
N_CHUNKS = N_PER_W // CHUNK
NBUF = 4
N_STEPS = N_CHUNKS // NBUF
GPC = CHUNK // 16
IBLK = 128
OBLK = 32


def _sc_body(ci_hbm, xi_hbm, in_emb_hbm, out_emb_hbm, out_hbm,
             istage, fci, fxi,
             cb0, xb0, cb1, xb1, cb2, xb2, cb3, xb3,
             scores, ostage, s0, s1, s2, s3):
    cbufs = (cb0, cb1, cb2, cb3)
    xbufs = (xb0, xb1, xb2, xb3)
    sems = (s0, s1, s2, s3)
    wid = lax.axis_index("s") * 2 + lax.axis_index("c")
    row_base = wid * ROWS_PER_W

    def load_indices(src_hbm, dst):
        for blk in range(ROWS_PER_W // IBLK):
            pltpu.sync_copy(
                src_hbm.at[pl.ds(row_base + blk * IBLK, IBLK)], istage)

            def row_body(r, carry):
                o = (blk * IBLK + r) * L
                dst[pl.ds(o, 16)] = istage[r, pl.ds(0, 16)]
                dst[pl.ds(o + 4, 16)] = istage[r, pl.ds(4, 16)]
                return carry
            lax.fori_loop(0, IBLK, row_body, 0)

    load_indices(ci_hbm, fci)
    load_indices(xi_hbm, fxi)

    iota16 = lax.iota(jnp.int32, 16)

    def fire(c, k):
        pltpu.async_copy(in_emb_hbm.at[fci.at[pl.ds(c * CHUNK, CHUNK)]],
                         cbufs[k], sems[k])
        pltpu.async_copy(out_emb_hbm.at[fxi.at[pl.ds(c * CHUNK, CHUNK)]],
                         xbufs[k], sems[k])

    def drain(c, k):
        pltpu.make_async_copy(in_emb_hbm.at[fci.at[pl.ds(c * CHUNK, CHUNK)]],
                              cbufs[k], sems[k]).wait()
        pltpu.make_async_copy(out_emb_hbm.at[fxi.at[pl.ds(c * CHUNK, CHUNK)]],
                              xbufs[k], sems[k]).wait()

    def compute(c, k):
        cbuf, xbuf = cbufs[k], xbufs[k]

        def group_body(g, carry):
            base = g * 16
            acc = jnp.zeros((16,), jnp.float32)
            for r in range(16):
                row = base + r
                s = (cbuf[row, pl.ds(0, 16)] * xbuf[row, pl.ds(0, 16)]
                     + cbuf[row, pl.ds(16, 16)] * xbuf[row, pl.ds(16, 16)]
                     + cbuf[row, pl.ds(32, 16)] * xbuf[row, pl.ds(32, 16)]
                     + cbuf[row, pl.ds(48, 16)] * xbuf[row, pl.ds(48, 16)])
                acc = jnp.where(iota16 == r, jnp.sum(s), acc)
            scores[pl.ds(c * CHUNK + base, 16)] = acc
            return carry

        lax.fori_loop(0, GPC, group_body, 0)

    for k in range(NBUF):
        fire(k, k)

    def step_body(i, carry):
        for k in range(NBUF):
            c = i * NBUF + k
            drain(c, k)
            compute(c, k)

            @pl.when(i < N_STEPS - 1)
            def _():
                fire(c + NBUF, k)
        return carry

    lax.fori_loop(0, N_STEPS, step_body, 0)

    for blk in range(ROWS_PER_W // OBLK):
        def orow_body(r, carry):
            o = (blk * OBLK + r) * L
            ostage[r, pl.ds(0, 16)] = scores[pl.ds(o, 16)]
            ostage[r, pl.ds(4, 16)] = scores[pl.ds(o + 4, 16)]
            return carry
        lax.fori_loop(0, OBLK, orow_body, 0)
        pltpu.sync_copy(ostage,
                        out_hbm.at[pl.ds(row_base + blk * OBLK, OBLK)])


def kernel(center_words, context_words, in_embeddings, out_embeddings):
    pad = ((0, 0), (0, LP - L))
    ci = jnp.pad(center_words.astype(jnp.int32), pad)
    xi = jnp.pad(context_words.astype(jnp.int32), pad)
    mesh = plsc.VectorSubcoreMesh(core_axis_name="c", subcore_axis_name="s")
    padded = pl.kernel(
        _sc_body,
        mesh=mesh,
        compiler_params=pltpu.CompilerParams(
            needs_layout_passes=False, use_tc_tiling_on_sc=False),
        out_type=jax.ShapeDtypeStruct((B, LP), jnp.float32),
        scratch_types=[
            pltpu.VMEM((IBLK, LP), jnp.int32),
            pltpu.VMEM((N_PER_W,), jnp.int32),
            pltpu.VMEM((N_PER_W,), jnp.int32),
            pltpu.VMEM((CHUNK, D), jnp.float32),
            pltpu.VMEM((CHUNK, D), jnp.float32),
            pltpu.VMEM((CHUNK, D), jnp.float32),
            pltpu.VMEM((CHUNK, D), jnp.float32),
            pltpu.VMEM((CHUNK, D), jnp.float32),
            pltpu.VMEM((CHUNK, D), jnp.float32),
            pltpu.VMEM((CHUNK, D), jnp.float32),
            pltpu.VMEM((CHUNK, D), jnp.float32),
            pltpu.VMEM((N_PER_W,), jnp.float32),
            pltpu.VMEM((OBLK, LP), jnp.float32),
            pltpu.SemaphoreType.DMA,
            pltpu.SemaphoreType.DMA,
            pltpu.SemaphoreType.DMA,
            pltpu.SemaphoreType.DMA,
        ],
    )(ci, xi, in_embeddings, out_embeddings)
    return padded[:, :L]


# speedup vs baseline: 1.0309x; 1.0283x over previous
import jax
import jax.numpy as jnp
from jax import lax
from jax.experimental import pallas as pl
from jax.experimental.pallas import tpu as pltpu
from jax.experimental.pallas import tpu_sc as plsc

B = 16384
L = 20
LP = 128
D = 64
N_TOTAL = B * L
NW = 32
ROWS_PER_W = B // NW
N_PER_W = N_TOTAL // NW
CHUNK = 64
N_CHUNKS = N_PER_W // CHUNK
NBUF = 4
N_STEPS = N_CHUNKS // NBUF
GPC = CHUNK // 16
IBLK = 128
OBLK = 32


def _sc_body(ci_hbm, xi_hbm, in_emb_hbm, out_emb_hbm, out_hbm,
             istage, fci, fxi,
             cb0, xb0, cb1, xb1, cb2, xb2, cb3, xb3,
             scores, ostage, s0, s1, s2, s3):
    cbufs = (cb0, cb1, cb2, cb3)
    xbufs = (xb0, xb1, xb2, xb3)
    sems = (s0, s1, s2, s3)
    wid = lax.axis_index("s") * 2 + lax.axis_index("c")
    row_base = wid * ROWS_PER_W

    def load_indices(src_hbm, dst):
        for blk in range(ROWS_PER_W // IBLK):
            pltpu.sync_copy(
                src_hbm.at[pl.ds(row_base + blk * IBLK, IBLK)], istage)

            def row_body(r, carry):
                o = (blk * IBLK + r) * L
                dst[pl.ds(o, 16)] = istage[r, pl.ds(0, 16)]
                dst[pl.ds(o + 4, 16)] = istage[r, pl.ds(4, 16)]
                return carry
            lax.fori_loop(0, IBLK, row_body, 0)

    load_indices(ci_hbm, fci)
    load_indices(xi_hbm, fxi)

    iota16 = lax.iota(jnp.int32, 16)

    def fire(c, k):
        pltpu.async_copy(in_emb_hbm.at[fci.at[pl.ds(c * CHUNK, CHUNK)]],
                         cbufs[k], sems[k])
        pltpu.async_copy(out_emb_hbm.at[fxi.at[pl.ds(c * CHUNK, CHUNK)]],
                         xbufs[k], sems[k])

    def drain(c, k):
        pltpu.make_async_copy(in_emb_hbm.at[fci.at[pl.ds(c * CHUNK, CHUNK)]],
                              cbufs[k], sems[k]).wait()
        pltpu.make_async_copy(out_emb_hbm.at[fxi.at[pl.ds(c * CHUNK, CHUNK)]],
                              xbufs[k], sems[k]).wait()

    def compute(c, k):
        cbuf, xbuf = cbufs[k], xbufs[k]

        def group_body(g, carry):
            base = g * 16
            acc = jnp.zeros((16,), jnp.float32)
            for r in range(16):
                row = base + r
                s = (cbuf[row, pl.ds(0, 16)] * xbuf[row, pl.ds(0, 16)]
                     + cbuf[row, pl.ds(16, 16)] * xbuf[row, pl.ds(16, 16)]
                     + cbuf[row, pl.ds(32, 16)] * xbuf[row, pl.ds(32, 16)]
                     + cbuf[row, pl.ds(48, 16)] * xbuf[row, pl.ds(48, 16)])
                acc = jnp.where(iota16 == r, jnp.sum(s), acc)
            scores[pl.ds(c * CHUNK + base, 16)] = acc
            return carry

        lax.fori_loop(0, GPC, group_body, 0)

    for k in range(NBUF):
        fire(k, k)

    def step_body(i, carry):
        for k in range(NBUF):
            c = i * NBUF + k
            drain(c, k)
            compute(c, k)

            @pl.when(i < N_STEPS - 1)
            def _():
                fire(c + NBUF, k)
        return carry

    lax.fori_loop(0, N_STEPS, step_body, 0)

    for blk in range(ROWS_PER_W // OBLK):
        def orow_body(r, carry):
            o = (blk * OBLK + r) * L
            ostage[r, pl.ds(0, 16)] = scores[pl.ds(o, 16)]
            ostage[r, pl.ds(4, 16)] = scores[pl.ds(o + 4, 16)]
            return carry
        lax.fori_loop(0, OBLK, orow_body, 0)
        pltpu.sync_copy(ostage,
                        out_hbm.at[pl.ds(row_base + blk * OBLK, OBLK)])


def kernel(center_words, context_words, in_embeddings, out_embeddings):
    pad = ((0, 0), (0, LP - L))
    ci = jnp.pad(center_words.astype(jnp.int32), pad)
    xi = jnp.pad(context_words.astype(jnp.int32), pad)
    tpad = ((0, 0), (0, LP - D))
    inp = jnp.pad(in_embeddings, tpad)
    outp = jnp.pad(out_embeddings, tpad)
    mesh = plsc.VectorSubcoreMesh(core_axis_name="c", subcore_axis_name="s")
    padded = pl.kernel(
        _sc_body,
        mesh=mesh,
        compiler_params=pltpu.CompilerParams(
            needs_layout_passes=False, use_tc_tiling_on_sc=False),
        out_type=jax.ShapeDtypeStruct((B, LP), jnp.float32),
        scratch_types=[
            pltpu.VMEM((IBLK, LP), jnp.int32),
            pltpu.VMEM((N_PER_W,), jnp.int32),
            pltpu.VMEM((N_PER_W,), jnp.int32),
            pltpu.VMEM((CHUNK, LP), jnp.float32),
            pltpu.VMEM((CHUNK, LP), jnp.float32),
            pltpu.VMEM((CHUNK, LP), jnp.float32),
            pltpu.VMEM((CHUNK, LP), jnp.float32),
            pltpu.VMEM((CHUNK, LP), jnp.float32),
            pltpu.VMEM((CHUNK, LP), jnp.float32),
            pltpu.VMEM((CHUNK, LP), jnp.float32),
            pltpu.VMEM((CHUNK, LP), jnp.float32),
            pltpu.VMEM((N_PER_W,), jnp.float32),
            pltpu.VMEM((OBLK, LP), jnp.float32),
            pltpu.SemaphoreType.DMA,
            pltpu.SemaphoreType.DMA,
            pltpu.SemaphoreType.DMA,
            pltpu.SemaphoreType.DMA,
        ],
    )(ci, xi, inp, outp)
    return padded[:, :L]


# speedup vs baseline: 1.0587x; 1.0269x over previous
import jax
import jax.numpy as jnp
from jax import lax
from jax.experimental import pallas as pl
from jax.experimental.pallas import tpu as pltpu
from jax.experimental.pallas import tpu_sc as plsc

B = 16384
L = 20
LP = 128
D = 64
N_TOTAL = B * L
NW = 32
COLS_PER_W = B // NW
N_PER_W = N_TOTAL // NW
CHUNK = 64
CPL = COLS_PER_W // CHUNK
N_CHUNKS = N_PER_W // CHUNK
NBUF = 4
N_STEPS = N_CHUNKS // NBUF
GPC = CHUNK // 16


def _sc_body(ci_hbm, xi_hbm, in_emb_hbm, out_emb_hbm, out_hbm,
             ci2d, xi2d,
             cb0, xb0, cb1, xb1, cb2, xb2, cb3, xb3,
             scores, s0, s1, s2, s3):
    cbufs = (cb0, cb1, cb2, cb3)
    xbufs = (xb0, xb1, xb2, xb3)
    sems = (s0, s1, s2, s3)
    wid = lax.axis_index("s") * 2 + lax.axis_index("c")
    col_base = wid * COLS_PER_W

    pltpu.sync_copy(ci_hbm.at[pl.ds(0, L), pl.ds(col_base, COLS_PER_W)], ci2d)
    pltpu.sync_copy(xi_hbm.at[pl.ds(0, L), pl.ds(col_base, COLS_PER_W)], xi2d)

    iota16 = lax.iota(jnp.int32, 16)

    def idx_slice(ref, c):
        return ref.at[c // CPL, pl.ds((c % CPL) * CHUNK, CHUNK)]

    def fire(c, k):
        pltpu.async_copy(in_emb_hbm.at[idx_slice(ci2d, c)], cbufs[k], sems[k])
        pltpu.async_copy(out_emb_hbm.at[idx_slice(xi2d, c)], xbufs[k], sems[k])

    def drain(c, k):
        pltpu.make_async_copy(in_emb_hbm.at[idx_slice(ci2d, c)],
                              cbufs[k], sems[k]).wait()
        pltpu.make_async_copy(out_emb_hbm.at[idx_slice(xi2d, c)],
                              xbufs[k], sems[k]).wait()

    def compute(c, k):
        cbuf, xbuf = cbufs[k], xbufs[k]
        row_l = c // CPL
        col0 = (c % CPL) * CHUNK

        def group_body(g, carry):
            base = g * 16
            acc = jnp.zeros((16,), jnp.float32)
            for r in range(16):
                row = base + r
                s = (cbuf[row, pl.ds(0, 16)] * xbuf[row, pl.ds(0, 16)]
                     + cbuf[row, pl.ds(16, 16)] * xbuf[row, pl.ds(16, 16)]
                     + cbuf[row, pl.ds(32, 16)] * xbuf[row, pl.ds(32, 16)]
                     + cbuf[row, pl.ds(48, 16)] * xbuf[row, pl.ds(48, 16)])
                acc = jnp.where(iota16 == r, jnp.sum(s), acc)
            scores[row_l, pl.ds(col0 + base, 16)] = acc
            return carry

        lax.fori_loop(0, GPC, group_body, 0)

    for k in range(NBUF):
        fire(k, k)

    def step_body(i, carry):
        for k in range(NBUF):
            c = i * NBUF + k
            drain(c, k)
            compute(c, k)

            @pl.when(i < N_STEPS - 1)
            def _():
                fire(c + NBUF, k)
        return carry

    lax.fori_loop(0, N_STEPS, step_body, 0)
    pltpu.sync_copy(scores,
                    out_hbm.at[pl.ds(0, L), pl.ds(col_base, COLS_PER_W)])


def kernel(center_words, context_words, in_embeddings, out_embeddings):
    ci = center_words.astype(jnp.int32).T
    xi = context_words.astype(jnp.int32).T
    tpad = ((0, 0), (0, LP - D))
    inp = jnp.pad(in_embeddings, tpad)
    outp = jnp.pad(out_embeddings, tpad)
    mesh = plsc.VectorSubcoreMesh(core_axis_name="c", subcore_axis_name="s")
    out_t = pl.kernel(
        _sc_body,
        mesh=mesh,
        compiler_params=pltpu.CompilerParams(
            needs_layout_passes=False, use_tc_tiling_on_sc=False),
        out_type=jax.ShapeDtypeStruct((L, B), jnp.float32),
        scratch_types=[
            pltpu.VMEM((L, COLS_PER_W), jnp.int32),
            pltpu.VMEM((L, COLS_PER_W), jnp.int32),
            pltpu.VMEM((CHUNK, LP), jnp.float32),
            pltpu.VMEM((CHUNK, LP), jnp.float32),
            pltpu.VMEM((CHUNK, LP), jnp.float32),
            pltpu.VMEM((CHUNK, LP), jnp.float32),
            pltpu.VMEM((CHUNK, LP), jnp.float32),
            pltpu.VMEM((CHUNK, LP), jnp.float32),
            pltpu.VMEM((CHUNK, LP), jnp.float32),
            pltpu.VMEM((CHUNK, LP), jnp.float32),
            pltpu.VMEM((L, COLS_PER_W), jnp.float32),
            pltpu.SemaphoreType.DMA,
            pltpu.SemaphoreType.DMA,
            pltpu.SemaphoreType.DMA,
            pltpu.SemaphoreType.DMA,
        ],
    )(ci, xi, inp, outp)
    return out_t.T


# speedup vs baseline: 1.0606x; 1.0018x over previous
import jax
import jax.numpy as jnp
from jax import lax
from jax.experimental import pallas as pl
from jax.experimental.pallas import tpu as pltpu
from jax.experimental.pallas import tpu_sc as plsc

B = 16384
L = 20
LP = 128
D = 64
N_TOTAL = B * L
NW = 32
COLS_PER_W = B // NW
N_PER_W = N_TOTAL // NW
CHUNK = 64
CPL = COLS_PER_W // CHUNK
N_CHUNKS = N_PER_W // CHUNK
NBUF = 4
N_STEPS = N_CHUNKS // NBUF
GPC = CHUNK // 16


def _sc_body(ci_hbm, xi_hbm, in_emb_hbm, out_emb_hbm, out_hbm,
             ci2d, xi2d,
             cb0, xb0, cb1, xb1, cb2, xb2, cb3, xb3,
             scores, s0, s1, s2, s3):
    cbufs = (cb0, cb1, cb2, cb3)
    xbufs = (xb0, xb1, xb2, xb3)
    sems = (s0, s1, s2, s3)
    wid = lax.axis_index("s") * 2 + lax.axis_index("c")
    col_base = wid * COLS_PER_W

    pltpu.sync_copy(ci_hbm.at[pl.ds(0, L), pl.ds(col_base, COLS_PER_W)], ci2d)
    pltpu.sync_copy(xi_hbm.at[pl.ds(0, L), pl.ds(col_base, COLS_PER_W)], xi2d)

    iota16 = lax.iota(jnp.int32, 16)

    def idx_slice(ref, c):
        return ref.at[c // CPL, pl.ds((c % CPL) * CHUNK, CHUNK)]

    def fire(c, k):
        pltpu.async_copy(in_emb_hbm.at[idx_slice(ci2d, c)], cbufs[k], sems[k])
        pltpu.async_copy(out_emb_hbm.at[idx_slice(xi2d, c)], xbufs[k], sems[k])

    def drain(c, k):
        pltpu.make_async_copy(in_emb_hbm.at[idx_slice(ci2d, c)],
                              cbufs[k], sems[k]).wait()
        pltpu.make_async_copy(out_emb_hbm.at[idx_slice(xi2d, c)],
                              xbufs[k], sems[k]).wait()

    def compute(c, k):
        cbuf, xbuf = cbufs[k], xbufs[k]
        row_l = c // CPL
        col0 = (c % CPL) * CHUNK

        def group_body(g, carry):
            base = g * 16
            acc = jnp.zeros((16,), jnp.float32)
            for r in range(16):
                row = base + r
                s = (cbuf[row, pl.ds(0, 16)] * xbuf[row, pl.ds(0, 16)]
                     + cbuf[row, pl.ds(16, 16)] * xbuf[row, pl.ds(16, 16)]
                     + cbuf[row, pl.ds(32, 16)] * xbuf[row, pl.ds(32, 16)]
                     + cbuf[row, pl.ds(48, 16)] * xbuf[row, pl.ds(48, 16)])
                acc = jnp.where(iota16 == r, jnp.sum(s), acc)
            scores[row_l, pl.ds(col0 + base, 16)] = acc
            return carry

        lax.fori_loop(0, GPC, group_body, 0)

    for k in range(NBUF):
        fire(k, k)

    def step_body(i, carry):
        for k in range(NBUF):
            c = i * NBUF + k
            drain(c, k)
            compute(c, k)

            @pl.when(i < N_STEPS - 1)
            def _():
                fire(c + NBUF, k)
        return carry

    lax.fori_loop(0, N_STEPS, step_body, 0)
    pltpu.sync_copy(scores,
                    out_hbm.at[pl.ds(0, L), pl.ds(col_base, COLS_PER_W)])


def kernel(center_words, context_words, in_embeddings, out_embeddings):
    ci = center_words.astype(jnp.int32).T
    xi = context_words.astype(jnp.int32).T
    tpad = ((0, 0), (0, LP - D))
    inp = jnp.pad(in_embeddings, tpad)
    outp = jnp.pad(out_embeddings.T, ((0, LP - D), (0, 0))).T
    mesh = plsc.VectorSubcoreMesh(core_axis_name="c", subcore_axis_name="s")
    out_t = pl.kernel(
        _sc_body,
        mesh=mesh,
        compiler_params=pltpu.CompilerParams(
            needs_layout_passes=False, use_tc_tiling_on_sc=False),
        out_type=jax.ShapeDtypeStruct((L, B), jnp.float32),
        scratch_types=[
            pltpu.VMEM((L, COLS_PER_W), jnp.int32),
            pltpu.VMEM((L, COLS_PER_W), jnp.int32),
            pltpu.VMEM((CHUNK, LP), jnp.float32),
            pltpu.VMEM((CHUNK, LP), jnp.float32),
            pltpu.VMEM((CHUNK, LP), jnp.float32),
            pltpu.VMEM((CHUNK, LP), jnp.float32),
            pltpu.VMEM((CHUNK, LP), jnp.float32),
            pltpu.VMEM((CHUNK, LP), jnp.float32),
            pltpu.VMEM((CHUNK, LP), jnp.float32),
            pltpu.VMEM((CHUNK, LP), jnp.float32),
            pltpu.VMEM((L, COLS_PER_W), jnp.float32),
            pltpu.SemaphoreType.DMA,
            pltpu.SemaphoreType.DMA,
            pltpu.SemaphoreType.DMA,
            pltpu.SemaphoreType.DMA,
        ],
    )(ci, xi, inp, outp)
    return out_t.T
